# baseline (speedup 1.0000x reference)
import functools

import jax
import jax.numpy as jnp
from jax import lax
from jax.experimental import pallas as pl
from jax.experimental.pallas import tpu as pltpu
from jax.experimental.pallas import tpu_sc as plsc

_SIGMA = 2.5
_H = 32
_K = 15
_CIN = 128
_COUT = 128

_NC = 2
_NS = 16
_NW = _NC * _NS
_CH = 128
_NB = 4


def _sc_gather(feats_tbl, sx, sy, sz, idx_flat):
    R = idx_flat.shape[0]
    N = feats_tbl.shape[0]
    per_w = R // _NW
    n_ch = per_w // _CH
    mesh = plsc.VectorSubcoreMesh(core_axis_name="c", subcore_axis_name="s")

    @functools.partial(
        pl.kernel,
        mesh=mesh,
        compiler_params=pltpu.CompilerParams(
            needs_layout_passes=False, use_tc_tiling_on_sc=False),
        out_type=(
            jax.ShapeDtypeStruct((R, _CIN // 2), jnp.int32),
            jax.ShapeDtypeStruct((2 * R,), jnp.float32),
            jax.ShapeDtypeStruct((2 * R,), jnp.float32),
            jax.ShapeDtypeStruct((2 * R,), jnp.float32),
        ),
        scratch_types=[
            pltpu.VMEM((per_w,), jnp.int32),
            [pltpu.VMEM((_CH, _CIN // 2), jnp.int32) for _ in range(_NB)],
            pltpu.VMEM((N,), jnp.float32),
            pltpu.VMEM((N,), jnp.float32),
            pltpu.VMEM((N,), jnp.float32),
            [pltpu.VMEM((2 * _CH,), jnp.float32) for _ in range(_NB)],
            [pltpu.VMEM((2 * _CH,), jnp.float32) for _ in range(_NB)],
            [pltpu.VMEM((2 * _CH,), jnp.float32) for _ in range(_NB)],
            pltpu.SemaphoreType.DMA,
            pltpu.SemaphoreType.DMA,
        ],
    )
    def gather_kernel(ftbl, xh, yh, zh, idx_hbm, fout, xout, yout, zout,
                      idx_v, fbufs, xtbl, ytbl, ztbl, xbufs, ybufs, zbufs,
                      gsem, wsem):
        wid = lax.axis_index("s") * _NC + lax.axis_index("c")
        base = wid * per_w
        pltpu.sync_copy(idx_hbm.at[pl.ds(base, per_w)], idx_v)
        pltpu.sync_copy(xh, xtbl)
        pltpu.sync_copy(yh, ytbl)
        pltpu.sync_copy(zh, ztbl)

        def fire(i, b):
            pltpu.async_copy(ftbl.at[idx_v.at[pl.ds(i * _CH, _CH)]], fbufs[b], gsem)

        for b in range(_NB):
            fire(b, b)

        def body(g, carry):
            for b in range(_NB):
                i = g + b
                off = base + i * _CH
                pltpu.make_async_copy(
                    ftbl.at[idx_v.at[pl.ds(i * _CH, _CH)]], fbufs[b], gsem).wait()
                lane2 = lax.iota(jnp.int32, 16) * 2
                for j in range(_CH // 16):
                    ir = idx_v[pl.ds(i * _CH + j * 16, 16)]
                    o2 = lane2 + (j * 32)
                    xv = plsc.load_gather(xtbl, [ir])
                    yv = plsc.load_gather(ytbl, [ir])
                    zv = plsc.load_gather(ztbl, [ir])
                    plsc.store_scatter(xbufs[b], [o2], xv)
                    plsc.store_scatter(xbufs[b], [o2 + 1], xv)
                    plsc.store_scatter(ybufs[b], [o2], yv)
                    plsc.store_scatter(ybufs[b], [o2 + 1], yv)
                    plsc.store_scatter(zbufs[b], [o2], zv)
                    plsc.store_scatter(zbufs[b], [o2 + 1], zv)

                pltpu.async_copy(fbufs[b], fout.at[pl.ds(off, _CH)], wsem)
                pltpu.async_copy(xbufs[b], xout.at[pl.ds(2 * off, 2 * _CH)], wsem)
                pltpu.async_copy(ybufs[b], yout.at[pl.ds(2 * off, 2 * _CH)], wsem)
                pltpu.async_copy(zbufs[b], zout.at[pl.ds(2 * off, 2 * _CH)], wsem)

                @pl.when(i + _NB < n_ch)
                def _():
                    pltpu.make_async_copy(fbufs[b], fout.at[pl.ds(off, _CH)], wsem).wait()
                    pltpu.make_async_copy(xbufs[b], xout.at[pl.ds(2 * off, 2 * _CH)], wsem).wait()
                    pltpu.make_async_copy(ybufs[b], yout.at[pl.ds(2 * off, 2 * _CH)], wsem).wait()
                    pltpu.make_async_copy(zbufs[b], zout.at[pl.ds(2 * off, 2 * _CH)], wsem).wait()
                    fire(i + _NB, b)
            return carry

        lax.fori_loop(0, n_ch // _NB, lambda t, c: body(t * _NB, c), 0, unroll=False)
        for b in range(_NB):
            i = n_ch - _NB + b
            off = base + i * _CH
            pltpu.make_async_copy(fbufs[b], fout.at[pl.ds(off, _CH)], wsem).wait()
            pltpu.make_async_copy(xbufs[b], xout.at[pl.ds(2 * off, 2 * _CH)], wsem).wait()
            pltpu.make_async_copy(ybufs[b], yout.at[pl.ds(2 * off, 2 * _CH)], wsem).wait()
            pltpu.make_async_copy(zbufs[b], zout.at[pl.ds(2 * off, 2 * _CH)], wsem).wait()

    return gather_kernel(feats_tbl, sx, sy, sz, idx_flat)


def _tc_body(f_ref, x_ref, y_ref, z_ref, q_ref, kp_ref, wf_ref, o_ref):
    ft = pltpu.bitcast(f_ref[...], jnp.bfloat16)
    q = q_ref[...]
    nx = x_ref[...] - q[:, 0:1]
    ny = y_ref[...] - q[:, 1:2]
    nz = z_ref[...] - q[:, 2:3]
    kp = kp_ref[...]
    dx = nx[:, None, :] - kp[0, :][None, :, None]
    dy = ny[:, None, :] - kp[1, :][None, :, None]
    dz = nz[:, None, :] - kp[2, :][None, :, None]
    d2 = dx * dx + dy * dy + dz * dz
    w3e = jnp.maximum(1.0 - jnp.sqrt(d2) * (1.0 / _SIGMA), 0.0)
    lane_j = lax.broadcasted_iota(jnp.int32, (1, 1, 2 * _H), 2) % 2
    w3_0 = jnp.where(lane_j == 0, w3e, 0.0).astype(jnp.bfloat16)
    w3_1 = jnp.where(lane_j == 0, 0.0, w3e).astype(jnp.bfloat16)
    wlo = lax.dot_general(
        w3_0, ft, (((2,), (1,)), ((0,), (0,))),
        preferred_element_type=jnp.float32)
    whi = lax.dot_general(
        w3_1, ft, (((2,), (1,)), ((0,), (0,))),
        preferred_element_type=jnp.float32)
    acc = None
    for k in range(_K):
        part = jnp.dot(wlo[:, k, :], wf_ref[k, :64, :],
                       preferred_element_type=jnp.float32)
        part = part + jnp.dot(whi[:, k, :], wf_ref[k, 64:, :],
                              preferred_element_type=jnp.float32)
        acc = part if acc is None else acc + part
    o_ref[...] = acc


def _tc_compute(gfeats, gx, gy, gz, q_pts, kernel_points, wflat, MP, B):
    grid = MP // B
    return pl.pallas_call(
        _tc_body,
        grid=(grid,),
        in_specs=[
            pl.BlockSpec((B, _H, _CIN // 2), lambda i: (i, 0, 0)),
            pl.BlockSpec((B, 2 * _H), lambda i: (i, 0)),
            pl.BlockSpec((B, 2 * _H), lambda i: (i, 0)),
            pl.BlockSpec((B, 2 * _H), lambda i: (i, 0)),
            pl.BlockSpec((B, 3), lambda i: (i, 0)),
            pl.BlockSpec((3, _K), lambda i: (0, 0)),
            pl.BlockSpec((_K, _CIN, _COUT), lambda i: (0, 0, 0)),
        ],
        out_specs=pl.BlockSpec((B, _COUT), lambda i: (i, 0)),
        out_shape=jax.ShapeDtypeStruct((MP, _COUT), jnp.float32),
    )(gfeats, gx, gy, gz, q_pts, kernel_points, wflat)


def kernel(q_pts, s_pts, s_feats, neighb_inds, weights, kernel_points):
    M = q_pts.shape[0]
    B = 256
    S = 10
    MP = ((M + S * B - 1) // (S * B)) * (S * B)
    MS = MP // S
    idx = neighb_inds.astype(jnp.int32)
    if MP != M:
        idx = jnp.pad(idx, ((0, MP - M), (0, 0)))
        q_pts = jnp.pad(q_pts, ((0, MP - M), (0, 0)))
    fb = s_feats.astype(jnp.bfloat16)
    fwords = jax.lax.bitcast_convert_type(
        jnp.stack([fb[:, :_CIN // 2], fb[:, _CIN // 2:]], axis=2), jnp.int32)
    sx, sy, sz = s_pts[:, 0], s_pts[:, 1], s_pts[:, 2]
    kpt = kernel_points.T
    outs = []
    for s in range(S):
        idx_flat = idx[s * MS:(s + 1) * MS].reshape(-1)
        gfeats, gx, gy, gz = _sc_gather(fwords, sx, sy, sz, idx_flat)
        outs.append(_tc_compute(
            gfeats.reshape(MS, _H, _CIN // 2),
            gx.reshape(MS, 2 * _H), gy.reshape(MS, 2 * _H), gz.reshape(MS, 2 * _H),
            q_pts[s * MS:(s + 1) * MS], kpt, weights, MS, B))
    out = jnp.concatenate(outs, axis=0)
    return out[:M]


# speedup vs baseline: 1.1779x; 1.1779x over previous
import functools

import jax
import jax.numpy as jnp
from jax import lax
from jax.experimental import pallas as pl
from jax.experimental.pallas import tpu as pltpu
from jax.experimental.pallas import tpu_sc as plsc

_SIGMA = 2.5
_H = 32
_K = 15
_CIN = 128
_COUT = 128

_NC = 2
_NS = 16
_NW = _NC * _NS
_CH = 128
_NB = 4


def _sc_gather(feats_tbl, sx, sy, sz, idx_flat):
    R = idx_flat.shape[0]
    N = feats_tbl.shape[0]
    per_w = R // _NW
    n_ch = per_w // _CH
    mesh = plsc.VectorSubcoreMesh(core_axis_name="c", subcore_axis_name="s")

    @functools.partial(
        pl.kernel,
        mesh=mesh,
        compiler_params=pltpu.CompilerParams(
            needs_layout_passes=False, use_tc_tiling_on_sc=False),
        out_type=(
            jax.ShapeDtypeStruct((R, _CIN // 2), jnp.int32),
            jax.ShapeDtypeStruct((2 * R,), jnp.float32),
            jax.ShapeDtypeStruct((2 * R,), jnp.float32),
            jax.ShapeDtypeStruct((2 * R,), jnp.float32),
        ),
        scratch_types=[
            pltpu.VMEM((per_w,), jnp.int32),
            [pltpu.VMEM((_CH, _CIN // 2), jnp.int32) for _ in range(_NB)],
            pltpu.VMEM((N,), jnp.float32),
            pltpu.VMEM((N,), jnp.float32),
            pltpu.VMEM((N,), jnp.float32),
            [pltpu.VMEM((2 * _CH,), jnp.float32) for _ in range(_NB)],
            [pltpu.VMEM((2 * _CH,), jnp.float32) for _ in range(_NB)],
            [pltpu.VMEM((2 * _CH,), jnp.float32) for _ in range(_NB)],
            pltpu.SemaphoreType.DMA,
            pltpu.SemaphoreType.DMA,
        ],
    )
    def gather_kernel(ftbl, xh, yh, zh, idx_hbm, fout, xout, yout, zout,
                      idx_v, fbufs, xtbl, ytbl, ztbl, xbufs, ybufs, zbufs,
                      gsem, wsem):
        wid = lax.axis_index("s") * _NC + lax.axis_index("c")
        base = wid * per_w
        pltpu.sync_copy(idx_hbm.at[pl.ds(base, per_w)], idx_v)
        pltpu.sync_copy(xh, xtbl)
        pltpu.sync_copy(yh, ytbl)
        pltpu.sync_copy(zh, ztbl)

        def fire(i, b):
            pltpu.async_copy(ftbl.at[idx_v.at[pl.ds(i * _CH, _CH)]], fbufs[b], gsem)

        for b in range(_NB):
            fire(b, b)

        def body(g, carry):
            for b in range(_NB):
                i = g + b
                off = base + i * _CH
                pltpu.make_async_copy(
                    ftbl.at[idx_v.at[pl.ds(i * _CH, _CH)]], fbufs[b], gsem).wait()
                lane2 = lax.iota(jnp.int32, 16) * 2
                for j in range(_CH // 16):
                    ir = idx_v[pl.ds(i * _CH + j * 16, 16)]
                    o2 = lane2 + (j * 32)
                    xv = plsc.load_gather(xtbl, [ir])
                    yv = plsc.load_gather(ytbl, [ir])
                    zv = plsc.load_gather(ztbl, [ir])
                    plsc.store_scatter(xbufs[b], [o2], xv)
                    plsc.store_scatter(xbufs[b], [o2 + 1], xv)
                    plsc.store_scatter(ybufs[b], [o2], yv)
                    plsc.store_scatter(ybufs[b], [o2 + 1], yv)
                    plsc.store_scatter(zbufs[b], [o2], zv)
                    plsc.store_scatter(zbufs[b], [o2 + 1], zv)

                pltpu.async_copy(fbufs[b], fout.at[pl.ds(off, _CH)], wsem)
                pltpu.async_copy(xbufs[b], xout.at[pl.ds(2 * off, 2 * _CH)], wsem)
                pltpu.async_copy(ybufs[b], yout.at[pl.ds(2 * off, 2 * _CH)], wsem)
                pltpu.async_copy(zbufs[b], zout.at[pl.ds(2 * off, 2 * _CH)], wsem)

                @pl.when(i + _NB < n_ch)
                def _():
                    pltpu.make_async_copy(fbufs[b], fout.at[pl.ds(off, _CH)], wsem).wait()
                    pltpu.make_async_copy(xbufs[b], xout.at[pl.ds(2 * off, 2 * _CH)], wsem).wait()
                    pltpu.make_async_copy(ybufs[b], yout.at[pl.ds(2 * off, 2 * _CH)], wsem).wait()
                    pltpu.make_async_copy(zbufs[b], zout.at[pl.ds(2 * off, 2 * _CH)], wsem).wait()
                    fire(i + _NB, b)
            return carry

        lax.fori_loop(0, n_ch // _NB, lambda t, c: body(t * _NB, c), 0, unroll=False)
        for b in range(_NB):
            i = n_ch - _NB + b
            off = base + i * _CH
            pltpu.make_async_copy(fbufs[b], fout.at[pl.ds(off, _CH)], wsem).wait()
            pltpu.make_async_copy(xbufs[b], xout.at[pl.ds(2 * off, 2 * _CH)], wsem).wait()
            pltpu.make_async_copy(ybufs[b], yout.at[pl.ds(2 * off, 2 * _CH)], wsem).wait()
            pltpu.make_async_copy(zbufs[b], zout.at[pl.ds(2 * off, 2 * _CH)], wsem).wait()

    return gather_kernel(feats_tbl, sx, sy, sz, idx_flat)


def _tc_body(f_ref, x_ref, y_ref, z_ref, q_ref, kp_ref, wf_ref, o_ref):
    ft = pltpu.bitcast(f_ref[...], jnp.bfloat16)
    q = q_ref[...]
    nx = x_ref[...] - q[:, 0:1]
    ny = y_ref[...] - q[:, 1:2]
    nz = z_ref[...] - q[:, 2:3]
    kp = kp_ref[...]
    dx = nx[:, None, :] - kp[0, :][None, :, None]
    dy = ny[:, None, :] - kp[1, :][None, :, None]
    dz = nz[:, None, :] - kp[2, :][None, :, None]
    d2 = dx * dx + dy * dy + dz * dz
    w3e = jnp.maximum(1.0 - jnp.sqrt(d2) * (1.0 / _SIGMA), 0.0)
    lane_j = lax.broadcasted_iota(jnp.int32, (1, 1, 2 * _H), 2) % 2
    w3_0 = jnp.where(lane_j == 0, w3e, 0.0).astype(jnp.bfloat16)
    w3_1 = jnp.where(lane_j == 0, 0.0, w3e).astype(jnp.bfloat16)
    wlo = lax.dot_general(
        w3_0, ft, (((2,), (1,)), ((0,), (0,))),
        preferred_element_type=jnp.float32)
    whi = lax.dot_general(
        w3_1, ft, (((2,), (1,)), ((0,), (0,))),
        preferred_element_type=jnp.float32)
    acc = None
    for k in range(_K):
        part = jnp.dot(wlo[:, k, :], wf_ref[k, :64, :],
                       preferred_element_type=jnp.float32)
        part = part + jnp.dot(whi[:, k, :], wf_ref[k, 64:, :],
                              preferred_element_type=jnp.float32)
        acc = part if acc is None else acc + part
    o_ref[...] = acc


def _tc_compute(gfeats, gx, gy, gz, q_pts, kernel_points, wflat, MP, B):
    grid = MP // B
    return pl.pallas_call(
        _tc_body,
        grid=(grid,),
        in_specs=[
            pl.BlockSpec((B, _H, _CIN // 2), lambda i: (i, 0, 0)),
            pl.BlockSpec((B, 2 * _H), lambda i: (i, 0)),
            pl.BlockSpec((B, 2 * _H), lambda i: (i, 0)),
            pl.BlockSpec((B, 2 * _H), lambda i: (i, 0)),
            pl.BlockSpec((B, 3), lambda i: (i, 0)),
            pl.BlockSpec((3, _K), lambda i: (0, 0)),
            pl.BlockSpec((_K, _CIN, _COUT), lambda i: (0, 0, 0)),
        ],
        out_specs=pl.BlockSpec((B, _COUT), lambda i: (i, 0)),
        out_shape=jax.ShapeDtypeStruct((MP, _COUT), jnp.float32),
    )(gfeats, gx, gy, gz, q_pts, kernel_points, wflat)


def kernel(q_pts, s_pts, s_feats, neighb_inds, weights, kernel_points):
    M = q_pts.shape[0]
    B = 256
    S = 5
    MP = ((M + S * B - 1) // (S * B)) * (S * B)
    MS = MP // S
    idx = neighb_inds.astype(jnp.int32)
    if MP != M:
        idx = jnp.pad(idx, ((0, MP - M), (0, 0)))
        q_pts = jnp.pad(q_pts, ((0, MP - M), (0, 0)))
    fb = s_feats.astype(jnp.bfloat16)
    fwords = jax.lax.bitcast_convert_type(
        jnp.stack([fb[:, :_CIN // 2], fb[:, _CIN // 2:]], axis=2), jnp.int32)
    sx, sy, sz = s_pts[:, 0], s_pts[:, 1], s_pts[:, 2]
    kpt = kernel_points.T
    outs = []
    for s in range(S):
        idx_flat = idx[s * MS:(s + 1) * MS].reshape(-1)
        gfeats, gx, gy, gz = _sc_gather(fwords, sx, sy, sz, idx_flat)
        outs.append(_tc_compute(
            gfeats.reshape(MS, _H, _CIN // 2),
            gx.reshape(MS, 2 * _H), gy.reshape(MS, 2 * _H), gz.reshape(MS, 2 * _H),
            q_pts[s * MS:(s + 1) * MS], kpt, weights, MS, B))
    out = jnp.concatenate(outs, axis=0)
    return out[:M]
